# baseline (speedup 1.0000x reference)
import functools

import jax
import jax.numpy as jnp
from jax import lax
from jax.experimental import pallas as pl
from jax.experimental.pallas import tpu as pltpu
from jax.experimental.pallas import tpu_sc as plsc

_MAXLEN = 8192
_DIM = 64
_BATCH = 64

_NC = 2
_NS = 16
_NW = _NC * _NS

_SLAB = _MAXLEN // _NW
_CHUNK = _SLAB * _DIM
_LANES = 16
_VITERS = _CHUNK // _LANES
_NBUF = 2


def _sc_body(x_hbm, pos_hbm, out_hbm,
             in_bufs, out_bufs, pos_buf,
             lsem0, lsem1, ssem0, ssem1):
    lsems = (lsem0, lsem1)
    ssems = (ssem0, ssem1)
    wid = lax.axis_index("s") * _NC + lax.axis_index("c")
    slab_off = pl.multiple_of(wid * _CHUNK, _CHUNK)

    pltpu.sync_copy(pos_hbm.at[pl.ds(slab_off, _CHUNK)], pos_buf)

    def chunk_off(b):
        return pl.multiple_of(b * (_MAXLEN * _DIM) + slab_off, _CHUNK)

    def load(b, k):
        pltpu.async_copy(x_hbm.at[pl.ds(chunk_off(b), _CHUNK)],
                         in_bufs.at[k], lsems[k])

    def wait_load(b, k):
        pltpu.make_async_copy(x_hbm.at[pl.ds(chunk_off(b), _CHUNK)],
                              in_bufs.at[k], lsems[k]).wait()

    def store(b, k):
        pltpu.async_copy(out_bufs.at[k],
                         out_hbm.at[pl.ds(chunk_off(b), _CHUNK)], ssems[k])

    def wait_store(b, k):
        pltpu.make_async_copy(out_bufs.at[k],
                              out_hbm.at[pl.ds(chunk_off(b), _CHUNK)],
                              ssems[k]).wait()

    for k in range(_NBUF):
        load(k, k)

    def step(t, carry):
        for k in range(_NBUF):
            b = t * _NBUF + k
            wait_load(b, k)

            @pl.when(t > 0)
            def _():
                wait_store(b - _NBUF, k)

            def vadd(i, c):
                sl = pl.ds(i * _LANES, _LANES)
                out_bufs[k, sl] = in_bufs[k, sl] + pos_buf[sl]
                return c

            lax.fori_loop(0, _VITERS, vadd, 0, unroll=4)

            store(b, k)

            @pl.when(t < (_BATCH // _NBUF) - 1)
            def _():
                load(b + _NBUF, k)
        return carry

    lax.fori_loop(0, _BATCH // _NBUF, step, 0)

    for k in range(_NBUF):
        wait_store(_BATCH - _NBUF + k, k)


_sc_call = pl.kernel(
    _sc_body,
    out_type=jax.ShapeDtypeStruct((_BATCH * _MAXLEN * _DIM,), jnp.float32),
    mesh=plsc.VectorSubcoreMesh(core_axis_name="c", subcore_axis_name="s"),
    scratch_types=[
        pltpu.VMEM((_NBUF, _CHUNK), jnp.float32),
        pltpu.VMEM((_NBUF, _CHUNK), jnp.float32),
        pltpu.VMEM((_CHUNK,), jnp.float32),
        pltpu.SemaphoreType.DMA,
        pltpu.SemaphoreType.DMA,
        pltpu.SemaphoreType.DMA,
        pltpu.SemaphoreType.DMA,
    ],
)


@jax.jit
def kernel(x, pos_table):
    out = _sc_call(x.reshape(-1), pos_table.reshape(-1))
    return out.reshape(x.shape)


# speedup vs baseline: 1.3669x; 1.3669x over previous
import functools

import jax
import jax.numpy as jnp
from jax import lax
from jax.experimental import pallas as pl
from jax.experimental.pallas import tpu as pltpu
from jax.experimental.pallas import tpu_sc as plsc

_MAXLEN = 8192
_DIM = 64
_BATCH = 64

_NC = 2
_NS = 16
_NW = _NC * _NS

_SLAB = _MAXLEN // _NW
_CHUNK = _SLAB * _DIM
_LANES = 16
_VITERS = _CHUNK // _LANES
_NBUF = 2


def _sc_body(x_hbm, pos_hbm, out_hbm,
             in_bufs, out_bufs, pos_buf,
             lsem0, lsem1, ssem0, ssem1):
    lsems = (lsem0, lsem1)
    ssems = (ssem0, ssem1)
    wid = lax.axis_index("s") * _NC + lax.axis_index("c")
    slab_off = pl.multiple_of(wid * _CHUNK, _CHUNK)

    pltpu.sync_copy(pos_hbm.at[pl.ds(slab_off, _CHUNK)], pos_buf)

    def chunk_off(b):
        return pl.multiple_of(b * (_MAXLEN * _DIM) + slab_off, _CHUNK)

    def load(b, k):
        pltpu.async_copy(x_hbm.at[pl.ds(chunk_off(b), _CHUNK)],
                         in_bufs.at[k], lsems[k])

    def wait_load(b, k):
        pltpu.make_async_copy(x_hbm.at[pl.ds(chunk_off(b), _CHUNK)],
                              in_bufs.at[k], lsems[k]).wait()

    def store(b, k):
        pltpu.async_copy(out_bufs.at[k],
                         out_hbm.at[pl.ds(chunk_off(b), _CHUNK)], ssems[k])

    def wait_store(b, k):
        pltpu.make_async_copy(out_bufs.at[k],
                              out_hbm.at[pl.ds(chunk_off(b), _CHUNK)],
                              ssems[k]).wait()

    for k in range(_NBUF):
        load(k, k)

    def step(t, carry):
        for k in range(_NBUF):
            b = t * _NBUF + k
            wait_load(b, k)

            @pl.when(t > 0)
            def _():
                wait_store(b - _NBUF, k)

            @plsc.parallel_loop(0, _CHUNK, step=_LANES, unroll=8)
            def _(i):
                sl = pl.ds(i, _LANES)
                out_bufs[k, sl] = in_bufs[k, sl] + pos_buf[sl]

            store(b, k)

            @pl.when(t < (_BATCH // _NBUF) - 1)
            def _():
                load(b + _NBUF, k)
        return carry

    lax.fori_loop(0, _BATCH // _NBUF, step, 0)

    for k in range(_NBUF):
        wait_store(_BATCH - _NBUF + k, k)


_sc_call = pl.kernel(
    _sc_body,
    out_type=jax.ShapeDtypeStruct((_BATCH * _MAXLEN * _DIM,), jnp.float32),
    mesh=plsc.VectorSubcoreMesh(core_axis_name="c", subcore_axis_name="s"),
    scratch_types=[
        pltpu.VMEM((_NBUF, _CHUNK), jnp.float32),
        pltpu.VMEM((_NBUF, _CHUNK), jnp.float32),
        pltpu.VMEM((_CHUNK,), jnp.float32),
        pltpu.SemaphoreType.DMA,
        pltpu.SemaphoreType.DMA,
        pltpu.SemaphoreType.DMA,
        pltpu.SemaphoreType.DMA,
    ],
)


@jax.jit
def kernel(x, pos_table):
    out = _sc_call(x.reshape(-1), pos_table.reshape(-1))
    return out.reshape(x.shape)


# speedup vs baseline: 1.3900x; 1.0169x over previous
import jax
import jax.numpy as jnp
import numpy as np
from jax import lax
from jax.experimental import pallas as pl
from jax.experimental.pallas import tpu as pltpu
from jax.experimental.pallas import tpu_sc as plsc

_MAXLEN = 8192
_DIM = 64
_BATCH = 64

_NC = 2
_NS = 16
_NW = _NC * _NS

_ROWW = 128
_PROWS = _MAXLEN * _DIM // _ROWW
_CROWS = _PROWS // _NW
_NCHUNK = _BATCH
_NBUF = 4


def _sc_body(x_hbm, pos_hbm, idx_hbm, out_hbm,
             bufs, idx_buf,
             lsem0, lsem1, lsem2, lsem3,
             gsem0, gsem1, gsem2, gsem3,
             ssem0, ssem1, ssem2, ssem3):
    lsems = (lsem0, lsem1, lsem2, lsem3)
    gsems = (gsem0, gsem1, gsem2, gsem3)
    ssems = (ssem0, ssem1, ssem2, ssem3)

    wid = lax.axis_index("s") * _NC + lax.axis_index("c")
    base_row = wid * _CROWS

    pltpu.sync_copy(idx_hbm.at[wid], idx_buf)

    def row0(c):
        return c * _PROWS + base_row

    def load(c, k):
        pltpu.async_copy(x_hbm.at[pl.ds(row0(c), _CROWS)], bufs.at[k],
                         lsems[k])

    def wait_load(c, k):
        pltpu.make_async_copy(x_hbm.at[pl.ds(row0(c), _CROWS)], bufs.at[k],
                              lsems[k]).wait()

    def gather_add(c, k):
        pltpu.async_copy(pos_hbm.at[idx_buf], bufs.at[k], gsems[k], add=True)

    def wait_gather(c, k):
        pltpu.make_async_copy(pos_hbm.at[idx_buf], bufs.at[k],
                              gsems[k]).wait()

    def store(c, k):
        pltpu.async_copy(bufs.at[k], out_hbm.at[pl.ds(row0(c), _CROWS)],
                         ssems[k])

    def wait_store(c, k):
        pltpu.make_async_copy(bufs.at[k], out_hbm.at[pl.ds(row0(c), _CROWS)],
                              ssems[k]).wait()

    load(0, 0)
    load(1, 1)

    def step(t, carry):
        for k in range(_NBUF):
            c = t * _NBUF + k
            wait_load(c, k)
            gather_add(c, k)
            wait_gather(c, k)
            store(c, k)
            if k < 2:
                @pl.when(t > 0)
                def _():
                    wait_store(c - 2, (k + 2) % _NBUF)

                load(c + 2, (k + 2) % _NBUF)
            else:
                @pl.when(t < _NCHUNK // _NBUF - 1)
                def _():
                    wait_store(c - 2, (k + 2) % _NBUF)
                    load(c + 2, (k + 2) % _NBUF)
        return carry

    lax.fori_loop(0, _NCHUNK // _NBUF, step, 0)

    for c in range(_NCHUNK - _NBUF, _NCHUNK):
        wait_store(c, c % _NBUF)


_sc_call = pl.kernel(
    _sc_body,
    out_type=jax.ShapeDtypeStruct((_BATCH * _PROWS, _ROWW), jnp.float32),
    mesh=plsc.VectorSubcoreMesh(core_axis_name="c", subcore_axis_name="s"),
    scratch_types=[
        pltpu.VMEM((_NBUF, _CROWS, _ROWW), jnp.float32),
        pltpu.VMEM((_CROWS,), jnp.int32),
        pltpu.SemaphoreType.DMA,
        pltpu.SemaphoreType.DMA,
        pltpu.SemaphoreType.DMA,
        pltpu.SemaphoreType.DMA,
        pltpu.SemaphoreType.DMA,
        pltpu.SemaphoreType.DMA,
        pltpu.SemaphoreType.DMA,
        pltpu.SemaphoreType.DMA,
        pltpu.SemaphoreType.DMA,
        pltpu.SemaphoreType.DMA,
        pltpu.SemaphoreType.DMA,
        pltpu.SemaphoreType.DMA,
    ],
)

_POS_IDX = np.arange(_PROWS, dtype=np.int32).reshape(_NW, _CROWS)


@jax.jit
def kernel(x, pos_table):
    out = _sc_call(x.reshape(_BATCH * _PROWS, _ROWW),
                   pos_table.reshape(_PROWS, _ROWW), _POS_IDX)
    return out.reshape(x.shape)


# speedup vs baseline: 2.6231x; 1.8871x over previous
import jax
import jax.numpy as jnp
import numpy as np
from jax import lax
from jax.experimental import pallas as pl
from jax.experimental.pallas import tpu as pltpu
from jax.experimental.pallas import tpu_sc as plsc

_MAXLEN = 8192
_DIM = 64
_BATCH = 64

_NC = 2
_NS = 16
_NW = _NC * _NS

_TR = 8
_PROWS = _MAXLEN // _TR
_SLAB = _PROWS // _NW
_CR = 16
_CPB = _SLAB // _CR
_NCHUNK = _BATCH * _CPB
_NBUF = 4
_LANES = 16


def _sc_body(x_hbm, pos_hbm, out_hbm,
             bufs, pos_buf,
             lsem0, lsem1, lsem2, lsem3,
             ssem0, ssem1, ssem2, ssem3):
    lsems = (lsem0, lsem1, lsem2, lsem3)
    ssems = (ssem0, ssem1, ssem2, ssem3)

    wid = lax.axis_index("s") * _NC + lax.axis_index("c")
    base_row = wid * _SLAB

    pltpu.sync_copy(pos_hbm.at[pl.ds(base_row, _SLAB)], pos_buf)

    def row0(c):
        b = c // _CPB
        j = lax.rem(c, _CPB)
        return b * _PROWS + base_row + j * _CR

    def load(c, k):
        pltpu.async_copy(x_hbm.at[pl.ds(row0(c), _CR)], bufs.at[k],
                         lsems[k])

    def wait_load(c, k):
        pltpu.make_async_copy(x_hbm.at[pl.ds(row0(c), _CR)], bufs.at[k],
                              lsems[k]).wait()

    def store(c, k):
        pltpu.async_copy(bufs.at[k], out_hbm.at[pl.ds(row0(c), _CR)],
                         ssems[k])

    def wait_store(c, k):
        pltpu.make_async_copy(bufs.at[k], out_hbm.at[pl.ds(row0(c), _CR)],
                              ssems[k]).wait()

    load(0, 0)
    load(1, 1)

    def step(t, carry):
        for k in range(_NBUF):
            c = t * _NBUF + k
            j = k % _CPB
            wait_load(c, k)

            @plsc.parallel_loop(0, _CR * _TR * _DIM, step=_LANES, unroll=8)
            def _(i):
                r = i // (_TR * _DIM)
                rem = lax.rem(i, _TR * _DIM)
                tt = rem // _DIM
                l = lax.rem(rem, _DIM)
                sl = pl.ds(l, _LANES)
                plsc.addupdate(bufs.at[k, r, tt, sl],
                               pos_buf[j * _CR + r, tt, sl])

            store(c, k)
            if k < 2:
                @pl.when(t > 0)
                def _():
                    wait_store(c - 2, (k + 2) % _NBUF)

                load(c + 2, (k + 2) % _NBUF)
            else:
                @pl.when(t < _NCHUNK // _NBUF - 1)
                def _():
                    wait_store(c - 2, (k + 2) % _NBUF)
                    load(c + 2, (k + 2) % _NBUF)
        return carry

    lax.fori_loop(0, _NCHUNK // _NBUF, step, 0)

    for c in range(_NCHUNK - _NBUF, _NCHUNK):
        wait_store(c, c % _NBUF)


_sc_call = pl.kernel(
    _sc_body,
    out_type=jax.ShapeDtypeStruct((_BATCH * _PROWS, _TR, _DIM), jnp.float32),
    mesh=plsc.VectorSubcoreMesh(core_axis_name="c", subcore_axis_name="s"),
    scratch_types=[
        pltpu.VMEM((_NBUF, _CR, _TR, _DIM), jnp.float32),
        pltpu.VMEM((_SLAB, _TR, _DIM), jnp.float32),
        pltpu.SemaphoreType.DMA,
        pltpu.SemaphoreType.DMA,
        pltpu.SemaphoreType.DMA,
        pltpu.SemaphoreType.DMA,
        pltpu.SemaphoreType.DMA,
        pltpu.SemaphoreType.DMA,
        pltpu.SemaphoreType.DMA,
        pltpu.SemaphoreType.DMA,
    ],
)


@jax.jit
def kernel(x, pos_table):
    out = _sc_call(x.reshape(_BATCH * _PROWS, _TR, _DIM),
                   pos_table.reshape(_PROWS, _TR, _DIM))
    return out.reshape(x.shape)
